# Initial kernel scaffold; baseline (speedup 1.0000x reference)
import jax
import jax.numpy as jnp
from jax.experimental import pallas as pl


def kernel(x, edge_index, edge_weight, W1, b1, W2, b2):
    raise NotImplementedError("write your pallas kernel here")


# speedup vs baseline: 3.0974x; 3.0974x over previous
import functools

import jax
import jax.numpy as jnp
from jax import lax
from jax.experimental import pallas as pl
from jax.experimental.pallas import tpu as pltpu
from jax.experimental.pallas import tpu_sc as plsc

N = 10000
D = 128
E = 320000
KCHEB = 8
NC, NS = 2, 16
NW = NC * NS
EPT = 10240
E_PAD = EPT * NW
G = 128
NCH = EPT // G
NROWS2D = E_PAD // 128
NPAD = 10240
NACC = 10240
RPT = NACC // NS

_mesh = plsc.VectorSubcoreMesh(core_axis_name="c", subcore_axis_name="s")
_sc_params = pltpu.CompilerParams(needs_layout_passes=False)


def _wid():
    return lax.axis_index("s") * NC + lax.axis_index("c")


def _deg_body(src_hbm, ew_hbm, out_hbm, srcb, ewb, degl):
    wid = _wid()
    pltpu.sync_copy(src_hbm.at[pl.ds(wid * NCH, NCH)], srcb)
    pltpu.sync_copy(ew_hbm.at[pl.ds(wid * NCH, NCH)], ewb)
    zv = jnp.zeros((16,), jnp.float32)

    def zbody(j, carry):
        degl[pl.ds(j * 16, 16)] = zv
        return carry

    lax.fori_loop(0, NPAD // 16, zbody, 0)

    def chunk(i, carry):
        for g in range(8):
            sl = pl.ds(g * 16, 16)
            idx = srcb[i, sl]
            vals = ewb[i, sl]
            plsc.addupdate_scatter(degl, [idx], vals)
        return carry

    lax.fori_loop(0, NCH, chunk, 0)
    pltpu.sync_copy(degl, out_hbm.at[pl.ds(wid * NPAD, NPAD)])


_deg_call = functools.partial(
    pl.kernel,
    out_type=jax.ShapeDtypeStruct((NW * NPAD,), jnp.float32),
    mesh=_mesh,
    compiler_params=_sc_params,
    scratch_types=[
        pltpu.VMEM((NCH, 128), jnp.int32),
        pltpu.VMEM((NCH, 128), jnp.float32),
        pltpu.VMEM((NPAD,), jnp.float32),
    ],
)(_deg_body)


def _dinv_body(degp_ref, dinv_ref):
    deg = jnp.sum(degp_ref[...], axis=0)
    dinv_ref[...] = jnp.where(deg > 0.0, lax.rsqrt(jnp.where(deg > 0.0, deg, 1.0)), 0.0)


def _dinv_call(degp):
    return pl.pallas_call(
        _dinv_body,
        out_shape=jax.ShapeDtypeStruct((NPAD // 128, 128), jnp.float32),
    )(degp)


def _norm_body(src_hbm, dst_hbm, ew_hbm, dinv_hbm, out_hbm, srcb, dstb, ewb, dinvv, normb):
    wid = _wid()
    pltpu.sync_copy(src_hbm.at[pl.ds(wid * NCH, NCH)], srcb)
    pltpu.sync_copy(dst_hbm.at[pl.ds(wid * NCH, NCH)], dstb)
    pltpu.sync_copy(ew_hbm.at[pl.ds(wid * NCH, NCH)], ewb)
    pltpu.sync_copy(dinv_hbm, dinvv)

    def chunk(i, carry):
        for g in range(8):
            sl = pl.ds(g * 16, 16)
            s = srcb[i, sl]
            d = dstb[i, sl]
            w = ewb[i, sl]
            dvs = plsc.load_gather(dinvv, [s])
            dvd = plsc.load_gather(dinvv, [d])
            normb[i, sl] = -(dvs * w * dvd)
        return carry

    lax.fori_loop(0, NCH, chunk, 0)
    pltpu.sync_copy(normb, out_hbm.at[pl.ds(wid * NCH, NCH)])


_norm_call = functools.partial(
    pl.kernel,
    out_type=jax.ShapeDtypeStruct((NROWS2D, 128), jnp.float32),
    mesh=_mesh,
    compiler_params=_sc_params,
    scratch_types=[
        pltpu.VMEM((NCH, 128), jnp.int32),
        pltpu.VMEM((NCH, 128), jnp.int32),
        pltpu.VMEM((NCH, 128), jnp.float32),
        pltpu.VMEM((NPAD,), jnp.float32),
        pltpu.VMEM((NCH, 128), jnp.float32),
    ],
)(_norm_body)


def _lhat_body(y_hbm, src_hbm, dst_hbm, norm_hbm, z_hbm, out_hbm,
               srcb, dstb, normv, rows, acc_sh, sem):
    cid = lax.axis_index("c")
    sid = lax.axis_index("s")
    wid = sid * NC + cid
    pltpu.sync_copy(z_hbm, acc_sh.at[pl.ds(sid * RPT, RPT)])
    pltpu.sync_copy(src_hbm.at[pl.ds(wid * NCH, NCH)], srcb)
    pltpu.sync_copy(dst_hbm.at[pl.ds(wid * NCH, NCH)], dstb)
    pltpu.sync_copy(norm_hbm.at[pl.ds(wid * EPT, EPT)], normv)
    plsc.subcore_barrier()

    def chunk(i, carry):
        pltpu.async_copy(y_hbm.at[srcb.at[i]], rows, sem).wait()

        def e16(u, c2):
            nv = normv[pl.ds(i * G + u * 16, 16)]
            for q in range(16):
                e = u * 16 + q
                sv = jnp.full((16,), nv[q], jnp.float32)
                for c in range(8):
                    sl = pl.ds(c * 16, 16)
                    rows[e, sl] = rows[e, sl] * sv
            return c2

        lax.fori_loop(0, G // 16, e16, 0)
        pltpu.sync_copy(rows, acc_sh.at[dstb.at[i]], add=True)
        return carry

    lax.fori_loop(0, NCH, chunk, 0)
    plsc.subcore_barrier()
    pltpu.sync_copy(acc_sh.at[pl.ds(sid * RPT, RPT)],
                    out_hbm.at[pl.ds(cid * NACC + sid * RPT, RPT)])


_lhat_call = functools.partial(
    pl.kernel,
    out_type=jax.ShapeDtypeStruct((NC * NACC, D), jnp.float32),
    mesh=_mesh,
    compiler_params=_sc_params,
    scratch_types=[
        pltpu.VMEM((NCH, 128), jnp.int32),
        pltpu.VMEM((NCH, 128), jnp.int32),
        pltpu.VMEM((EPT,), jnp.float32),
        pltpu.VMEM((G, D), jnp.float32),
        pltpu.VMEM_SHARED((NACC, D), jnp.float32),
        pltpu.SemaphoreType.DMA,
    ],
)(_lhat_body)


BR = 1000


def _tc_first_body(x_ref, p_ref, w0_ref, w1_ref, b_ref, acc_ref, t1_ref):
    t1 = p_ref[0] + p_ref[1]
    acc = jnp.dot(x_ref[...], w0_ref[...], preferred_element_type=jnp.float32)
    acc = acc + jnp.dot(t1, w1_ref[...], preferred_element_type=jnp.float32)
    acc_ref[...] = acc + b_ref[...]
    t1_ref[...] = t1


def _tc_step_body(p_ref, tpp_ref, w_ref, accin_ref, accout_ref, tk_ref):
    tk = 2.0 * (p_ref[0] + p_ref[1]) - tpp_ref[...]
    accout_ref[...] = accin_ref[...] + jnp.dot(
        tk, w_ref[...], preferred_element_type=jnp.float32)
    tk_ref[...] = tk


def _tc_last_body(p_ref, tpp_ref, w_ref, accin_ref, h_ref):
    tk = 2.0 * (p_ref[0] + p_ref[1]) - tpp_ref[...]
    h_ref[...] = jnp.maximum(
        accin_ref[...] + jnp.dot(tk, w_ref[...], preferred_element_type=jnp.float32),
        0.0)


_row_spec = pl.BlockSpec((BR, D), lambda i: (i, 0))
_p_spec = pl.BlockSpec((2, BR, D), lambda i: (0, i, 0))
_w_spec = pl.BlockSpec((D, D), lambda i: (0, 0))
_b_spec = pl.BlockSpec((1, D), lambda i: (0, 0))
_GRID = (N // BR,)


def _first_call(x, p, w0, w1, b):
    return pl.pallas_call(
        _tc_first_body,
        grid=_GRID,
        in_specs=[_row_spec, _p_spec, _w_spec, _w_spec, _b_spec],
        out_specs=[_row_spec, _row_spec],
        out_shape=[jax.ShapeDtypeStruct((N, D), jnp.float32),
                   jax.ShapeDtypeStruct((N, D), jnp.float32)],
    )(x, p, w0, w1, b)


def _step_call(p, tpp, w, accin):
    return pl.pallas_call(
        _tc_step_body,
        grid=_GRID,
        in_specs=[_p_spec, _row_spec, _w_spec, _row_spec],
        out_specs=[_row_spec, _row_spec],
        out_shape=[jax.ShapeDtypeStruct((N, D), jnp.float32),
                   jax.ShapeDtypeStruct((N, D), jnp.float32)],
    )(p, tpp, w, accin)


def _last_call(p, tpp, w, accin):
    return pl.pallas_call(
        _tc_last_body,
        grid=_GRID,
        in_specs=[_p_spec, _row_spec, _w_spec, _row_spec],
        out_specs=_row_spec,
        out_shape=jax.ShapeDtypeStruct((N, D), jnp.float32),
    )(p, tpp, w, accin)


def kernel(x, edge_index, edge_weight, W1, b1, W2, b2):
    src = edge_index[0].astype(jnp.int32)
    dst = edge_index[1].astype(jnp.int32)
    pad = E_PAD - E
    src_p = jnp.concatenate([src, jnp.zeros((pad,), jnp.int32)])
    dst_p = jnp.concatenate([dst, jnp.zeros((pad,), jnp.int32)])
    ew_p = jnp.concatenate([edge_weight.astype(jnp.float32),
                            jnp.zeros((pad,), jnp.float32)])
    src2d = src_p.reshape(NROWS2D, 128)
    dst2d = dst_p.reshape(NROWS2D, 128)
    ew2d = ew_p.reshape(NROWS2D, 128)

    degp = _deg_call(src2d, ew2d)
    dinv2d = _dinv_call(degp.reshape(NW, NPAD // 128, 128))
    norm2d = _norm_call(src2d, dst2d, ew2d, dinv2d.reshape(-1))
    normf = norm2d.reshape(-1)
    zeros = jnp.zeros((RPT, D), jnp.float32)

    def lhat_partials(y):
        p = _lhat_call(y, src2d, dst2d, normf, zeros)
        return p.reshape(NC, NACC, D)

    def layer(y, W, b):
        p = lhat_partials(y)
        acc, tprev = _first_call(y, p, W[0], W[1], b.reshape(1, D))
        tpp = y
        for k in range(2, KCHEB - 1):
            p = lhat_partials(tprev)
            acc, tk = _step_call(p, tpp, W[k], acc)
            tpp, tprev = tprev, tk
        p = lhat_partials(tprev)
        return _last_call(p, tpp, W[KCHEB - 1], acc)

    h = layer(x, W1, b1)
    return layer(h, W2, b2)


# speedup vs baseline: 3.3909x; 1.0947x over previous
import functools

import jax
import jax.numpy as jnp
from jax import lax
from jax.experimental import pallas as pl
from jax.experimental.pallas import tpu as pltpu
from jax.experimental.pallas import tpu_sc as plsc

N = 10000
D = 128
E = 320000
KCHEB = 8
NC, NS = 2, 16
NW = NC * NS
EPT = 10240
E_PAD = EPT * NW
G = 128
NCH = EPT // G
NROWS2D = E_PAD // 128
NPAD = 10240
NACC = 10240
RPT = NACC // NS

_mesh = plsc.VectorSubcoreMesh(core_axis_name="c", subcore_axis_name="s")
_sc_params = pltpu.CompilerParams(needs_layout_passes=False)


def _wid():
    return lax.axis_index("s") * NC + lax.axis_index("c")


def _deg_body(src_hbm, ew_hbm, out_hbm, srcb, ewb, degl):
    wid = _wid()
    pltpu.sync_copy(src_hbm.at[pl.ds(wid * NCH, NCH)], srcb)
    pltpu.sync_copy(ew_hbm.at[pl.ds(wid * NCH, NCH)], ewb)
    zv = jnp.zeros((16,), jnp.float32)

    def zbody(j, carry):
        degl[pl.ds(j * 16, 16)] = zv
        return carry

    lax.fori_loop(0, NPAD // 16, zbody, 0)

    def chunk(i, carry):
        for g in range(8):
            sl = pl.ds(g * 16, 16)
            idx = srcb[i, sl]
            vals = ewb[i, sl]
            plsc.addupdate_scatter(degl, [idx], vals)
        return carry

    lax.fori_loop(0, NCH, chunk, 0)
    pltpu.sync_copy(degl, out_hbm.at[pl.ds(wid * NPAD, NPAD)])


_deg_call = functools.partial(
    pl.kernel,
    out_type=jax.ShapeDtypeStruct((NW * NPAD,), jnp.float32),
    mesh=_mesh,
    compiler_params=_sc_params,
    scratch_types=[
        pltpu.VMEM((NCH, 128), jnp.int32),
        pltpu.VMEM((NCH, 128), jnp.float32),
        pltpu.VMEM((NPAD,), jnp.float32),
    ],
)(_deg_body)


def _dinv_body(degp_ref, dinv_ref):
    deg = jnp.sum(degp_ref[...], axis=0)
    dinv_ref[...] = jnp.where(deg > 0.0, lax.rsqrt(jnp.where(deg > 0.0, deg, 1.0)), 0.0)


def _dinv_call(degp):
    return pl.pallas_call(
        _dinv_body,
        out_shape=jax.ShapeDtypeStruct((NPAD // 128, 128), jnp.float32),
    )(degp)


def _norm_body(src_hbm, dst_hbm, ew_hbm, dinv_hbm, out_hbm, srcb, dstb, ewb, dinvv, normb):
    wid = _wid()
    pltpu.sync_copy(src_hbm.at[pl.ds(wid * NCH, NCH)], srcb)
    pltpu.sync_copy(dst_hbm.at[pl.ds(wid * NCH, NCH)], dstb)
    pltpu.sync_copy(ew_hbm.at[pl.ds(wid * NCH, NCH)], ewb)
    pltpu.sync_copy(dinv_hbm, dinvv)

    def chunk(i, carry):
        for g in range(8):
            sl = pl.ds(g * 16, 16)
            s = srcb[i, sl]
            d = dstb[i, sl]
            w = ewb[i, sl]
            dvs = plsc.load_gather(dinvv, [s])
            dvd = plsc.load_gather(dinvv, [d])
            normb[i, sl] = -(dvs * w * dvd)
        return carry

    lax.fori_loop(0, NCH, chunk, 0)
    pltpu.sync_copy(normb, out_hbm.at[pl.ds(wid * NCH, NCH)])


_norm_call = functools.partial(
    pl.kernel,
    out_type=jax.ShapeDtypeStruct((NROWS2D, 128), jnp.float32),
    mesh=_mesh,
    compiler_params=_sc_params,
    scratch_types=[
        pltpu.VMEM((NCH, 128), jnp.int32),
        pltpu.VMEM((NCH, 128), jnp.int32),
        pltpu.VMEM((NCH, 128), jnp.float32),
        pltpu.VMEM((NPAD,), jnp.float32),
        pltpu.VMEM((NCH, 128), jnp.float32),
    ],
)(_norm_body)


NBUF = 2
NEB = 6
NCHT = E_PAD // G


def _lhat_body(y_hbm, edata_hbm, norm_hbm, z_hbm, out_hbm, rows, edatav, normr,
               acc_sh, *sems):
    sem_g = sems[:NBUF]
    sem_s = sems[NBUF:2 * NBUF]
    sem_e = sems[2 * NBUF:2 * NBUF + NEB]
    sem_n = sems[2 * NBUF + NEB:]
    cid = lax.axis_index("c")
    sid = lax.axis_index("s")
    wid = sid * NC + cid
    base = wid * NCH
    pltpu.sync_copy(z_hbm, acc_sh.at[pl.ds(sid * RPT, RPT)])
    plsc.subcore_barrier()

    def start_edata(j, s):
        pltpu.async_copy(edata_hbm.at[base + j], edatav.at[s], sem_e[s])
        pltpu.async_copy(norm_hbm.at[base + j], normr.at[s], sem_n[s])

    def wait_edata(s):
        pltpu.make_async_copy(edata_hbm.at[base], edatav.at[s], sem_e[s]).wait()
        pltpu.make_async_copy(norm_hbm.at[base], normr.at[s], sem_n[s]).wait()

    def start_gather(j_slot, b):
        pltpu.async_copy(y_hbm.at[edatav.at[j_slot, 0]], rows.at[b], sem_g[b])

    def wait_gather(b):
        pltpu.make_async_copy(y_hbm.at[edatav.at[0, 0]], rows.at[b], sem_g[b]).wait()

    def start_scatter(j_slot, b):
        pltpu.async_copy(rows.at[b], acc_sh.at[edatav.at[j_slot, 1]], sem_s[b],
                         add=True)

    def wait_scatter(b):
        pltpu.make_async_copy(rows.at[b], acc_sh.at[edatav.at[0, 1]],
                              sem_s[b]).wait()

    def scale(j_slot, b):
        def e16(u, c2):
            nv = normr[j_slot, pl.ds(u * 16, 16)]
            for q in range(16):
                e = u * 16 + q
                sv = jnp.full((16,), nv[q], jnp.float32)
                for c in range(8):
                    sl = pl.ds(c * 16, 16)
                    rows[b, e, sl] = rows[b, e, sl] * sv
            return c2

        lax.fori_loop(0, G // 16, e16, 0)

    for s in range(NEB):
        start_edata(s, s)
    wait_edata(0)
    start_gather(0, 0)

    def body(j, b, eb, do_swait, do_g, do_e):
        wait_gather(b)
        if do_swait:
            wait_scatter(1 - b)
        if do_e:
            start_edata(j - 1 + NEB, (eb - 1) % NEB)
        if do_g:
            ebn = (eb + 1) % NEB
            wait_edata(ebn)
            start_gather(ebn, 1 - b)
        scale(eb, b)
        start_scatter(eb, b)

    body(0, 0, 0, False, True, False)

    def outer(o, carry):
        for q in range(NEB):
            j = 1 + o * NEB + q
            body(j, (1 + q) % NBUF, (1 + q) % NEB, True, True, True)
        return carry

    n_main = (NCH - 1 - (NEB + 1)) // NEB
    lax.fori_loop(0, n_main, outer, 0)
    for t in range(NEB + 1):
        j = 1 + n_main * NEB + t
        body(j, j % NBUF, j % NEB, True, j + 1 < NCH, j - 1 + NEB < NCH)
    wait_scatter((NCH - 1) % NBUF)
    plsc.subcore_barrier()
    pltpu.sync_copy(acc_sh.at[pl.ds(sid * RPT, RPT)],
                    out_hbm.at[pl.ds(cid * NACC + sid * RPT, RPT)])


_lhat_call = functools.partial(
    pl.kernel,
    out_type=jax.ShapeDtypeStruct((NC * NACC, D), jnp.float32),
    mesh=_mesh,
    compiler_params=_sc_params,
    scratch_types=[
        pltpu.VMEM((NBUF, G, D), jnp.float32),
        pltpu.VMEM((NEB, 2, 128), jnp.int32),
        pltpu.VMEM((NEB, 128), jnp.float32),
        pltpu.VMEM_SHARED((NACC, D), jnp.float32),
    ] + [pltpu.SemaphoreType.DMA] * (2 * NBUF + 2 * NEB),
)(_lhat_body)


BR = 1000


def _tc_first_body(x_ref, p_ref, w0_ref, w1_ref, b_ref, acc_ref, t1_ref):
    t1 = p_ref[0] + p_ref[1]
    acc = jnp.dot(x_ref[...], w0_ref[...], preferred_element_type=jnp.float32)
    acc = acc + jnp.dot(t1, w1_ref[...], preferred_element_type=jnp.float32)
    acc_ref[...] = acc + b_ref[...]
    t1_ref[...] = t1


def _tc_step_body(p_ref, tpp_ref, w_ref, accin_ref, accout_ref, tk_ref):
    tk = 2.0 * (p_ref[0] + p_ref[1]) - tpp_ref[...]
    accout_ref[...] = accin_ref[...] + jnp.dot(
        tk, w_ref[...], preferred_element_type=jnp.float32)
    tk_ref[...] = tk


def _tc_last_body(p_ref, tpp_ref, w_ref, accin_ref, h_ref):
    tk = 2.0 * (p_ref[0] + p_ref[1]) - tpp_ref[...]
    h_ref[...] = jnp.maximum(
        accin_ref[...] + jnp.dot(tk, w_ref[...], preferred_element_type=jnp.float32),
        0.0)


_row_spec = pl.BlockSpec((BR, D), lambda i: (i, 0))
_p_spec = pl.BlockSpec((2, BR, D), lambda i: (0, i, 0))
_w_spec = pl.BlockSpec((D, D), lambda i: (0, 0))
_b_spec = pl.BlockSpec((1, D), lambda i: (0, 0))
_GRID = (N // BR,)


def _first_call(x, p, w0, w1, b):
    return pl.pallas_call(
        _tc_first_body,
        grid=_GRID,
        in_specs=[_row_spec, _p_spec, _w_spec, _w_spec, _b_spec],
        out_specs=[_row_spec, _row_spec],
        out_shape=[jax.ShapeDtypeStruct((N, D), jnp.float32),
                   jax.ShapeDtypeStruct((N, D), jnp.float32)],
    )(x, p, w0, w1, b)


def _step_call(p, tpp, w, accin):
    return pl.pallas_call(
        _tc_step_body,
        grid=_GRID,
        in_specs=[_p_spec, _row_spec, _w_spec, _row_spec],
        out_specs=[_row_spec, _row_spec],
        out_shape=[jax.ShapeDtypeStruct((N, D), jnp.float32),
                   jax.ShapeDtypeStruct((N, D), jnp.float32)],
    )(p, tpp, w, accin)


def _last_call(p, tpp, w, accin):
    return pl.pallas_call(
        _tc_last_body,
        grid=_GRID,
        in_specs=[_p_spec, _row_spec, _w_spec, _row_spec],
        out_specs=_row_spec,
        out_shape=jax.ShapeDtypeStruct((N, D), jnp.float32),
    )(p, tpp, w, accin)


def kernel(x, edge_index, edge_weight, W1, b1, W2, b2):
    src = edge_index[0].astype(jnp.int32)
    dst = edge_index[1].astype(jnp.int32)
    pad = E_PAD - E
    src_p = jnp.concatenate([src, jnp.zeros((pad,), jnp.int32)])
    dst_p = jnp.concatenate([dst, jnp.zeros((pad,), jnp.int32)])
    ew_p = jnp.concatenate([edge_weight.astype(jnp.float32),
                            jnp.zeros((pad,), jnp.float32)])
    src2d = src_p.reshape(NROWS2D, 128)
    dst2d = dst_p.reshape(NROWS2D, 128)
    ew2d = ew_p.reshape(NROWS2D, 128)

    degp = _deg_call(src2d, ew2d)
    dinv2d = _dinv_call(degp.reshape(NW, NPAD // 128, 128))
    norm2d = _norm_call(src2d, dst2d, ew2d, dinv2d.reshape(-1))
    edata = jnp.stack([src2d, dst2d], axis=1)
    zeros = jnp.zeros((RPT, D), jnp.float32)

    def lhat_partials(y):
        p = _lhat_call(y, edata, norm2d, zeros)
        return p.reshape(NC, NACC, D)

    def layer(y, W, b):
        p = lhat_partials(y)
        acc, tprev = _first_call(y, p, W[0], W[1], b.reshape(1, D))
        tpp = y
        for k in range(2, KCHEB - 1):
            p = lhat_partials(tprev)
            acc, tk = _step_call(p, tpp, W[k], acc)
            tpp, tprev = tprev, tk
        p = lhat_partials(tprev)
        return _last_call(p, tpp, W[KCHEB - 1], acc)

    h = layer(x, W1, b1)
    return layer(h, W2, b2)
